# Initial kernel scaffold; baseline (speedup 1.0000x reference)
import jax
import jax.numpy as jnp
from jax.experimental import pallas as pl


def kernel(x, edge_index, batch, W1, b1, W2, b2, W3, b3, W4, b4, W5, b5, Wh1, bh1, Wh2, bh2):
    raise NotImplementedError("write your pallas kernel here")


# speedup vs baseline: 12.9921x; 12.9921x over previous
import functools

import jax
import jax.numpy as jnp
from jax import lax
from jax.experimental import pallas as pl
from jax.experimental.pallas import tpu as pltpu
from jax.experimental.pallas import tpu_sc as plsc

N = 10000
E = 320000
H = 128
G = 64

NC = 2
NS = 16
NW = NC * NS
EPW = E // NW
C = 80
CH = EPW // C
RPT = 624
TAIL = N - NS * RPT
ZR = 48
CW = 128

_mesh = plsc.VectorSubcoreMesh(core_axis_name="c", subcore_axis_name="s")


def _sc_degree(ei):
    @functools.partial(
        pl.kernel,
        out_type=jax.ShapeDtypeStruct((NC, N, CW), jnp.float32),
        mesh=_mesh,
        scratch_types=[
            pltpu.VMEM((CH, C), jnp.int32),
            pltpu.VMEM((C, CW), jnp.float32),
            pltpu.VMEM((ZR, CW), jnp.float32),
            pltpu.VMEM_SHARED((N, CW), jnp.float32),
        ],
    )
    def k(ei_hbm, out_hbm, didx, ones_v, zbuf, acc):
        c = lax.axis_index("c")
        s = lax.axis_index("s")
        wid = c * NS + s
        one16 = jnp.ones((16,), jnp.float32)
        zero16 = jnp.zeros((16,), jnp.float32)

        def fill(i, _):
            for j in range(CW // 16):
                ones_v[i, pl.ds(j * 16, 16)] = one16
            return 0

        lax.fori_loop(0, C, fill, 0)

        def zrow(i, _):
            for j in range(CW // 16):
                zbuf[i, pl.ds(j * 16, 16)] = zero16
            return 0

        lax.fori_loop(0, ZR, zrow, 0)
        for r in range(RPT // ZR):
            pltpu.sync_copy(zbuf, acc.at[pl.ds(s * RPT + r * ZR, ZR)])

        @pl.when(s == NS - 1)
        def _():
            pltpu.sync_copy(zbuf.at[pl.ds(0, TAIL)],
                            acc.at[pl.ds(NS * RPT, TAIL)])

        pltpu.sync_copy(ei_hbm.at[1, wid], didx)
        plsc.subcore_barrier()

        def body(j, _):
            pltpu.sync_copy(ones_v, acc.at[didx.at[j]], add=True)
            return 0

        lax.fori_loop(0, CH, body, 0)
        plsc.subcore_barrier()
        pltpu.sync_copy(acc.at[pl.ds(s * RPT, RPT)],
                        out_hbm.at[c, pl.ds(s * RPT, RPT)])

        @pl.when(s == NS - 1)
        def _():
            pltpu.sync_copy(acc.at[pl.ds(NS * RPT, TAIL)],
                            out_hbm.at[c, pl.ds(NS * RPT, TAIL)])

    return k(ei)


def _sc_scatter(y, ei):
    @functools.partial(
        pl.kernel,
        out_type=jax.ShapeDtypeStruct((NC, N, H), jnp.float32),
        mesh=_mesh,
        scratch_types=[
            pltpu.VMEM((CH, C), jnp.int32),
            pltpu.VMEM((CH, C), jnp.int32),
            pltpu.VMEM((C, H), jnp.float32),
            pltpu.VMEM((ZR, H), jnp.float32),
            pltpu.VMEM_SHARED((N, H), jnp.float32),
            pltpu.SemaphoreType.DMA,
        ],
    )
    def k(y_hbm, ei_hbm, out_hbm, sidx, didx, rows, zbuf, acc, sem):
        c = lax.axis_index("c")
        s = lax.axis_index("s")
        wid = c * NS + s
        zero16 = jnp.zeros((16,), jnp.float32)

        def zrow(i, _):
            for j in range(H // 16):
                zbuf[i, pl.ds(j * 16, 16)] = zero16
            return 0

        lax.fori_loop(0, ZR, zrow, 0)
        for r in range(RPT // ZR):
            pltpu.sync_copy(zbuf, acc.at[pl.ds(s * RPT + r * ZR, ZR)])

        @pl.when(s == NS - 1)
        def _():
            pltpu.sync_copy(zbuf.at[pl.ds(0, TAIL)],
                            acc.at[pl.ds(NS * RPT, TAIL)])

        pltpu.sync_copy(ei_hbm.at[0, wid], sidx)
        pltpu.sync_copy(ei_hbm.at[1, wid], didx)
        plsc.subcore_barrier()

        def body(j, _):
            pltpu.async_copy(y_hbm.at[sidx.at[j]], rows, sem).wait()
            pltpu.sync_copy(rows, acc.at[didx.at[j]], add=True)
            return 0

        lax.fori_loop(0, CH, body, 0)
        plsc.subcore_barrier()
        pltpu.sync_copy(acc.at[pl.ds(s * RPT, RPT)],
                        out_hbm.at[c, pl.ds(s * RPT, RPT)])

        @pl.when(s == NS - 1)
        def _():
            pltpu.sync_copy(acc.at[pl.ds(NS * RPT, TAIL)],
                            out_hbm.at[c, pl.ds(NS * RPT, TAIL)])

    return k(y, ei)


_R = 2000


def _tc_prologue(x, W1, cnt0, cnt1):
    def body(x_ref, w_ref, c0_ref, c1_ref, y_ref, dis_ref):
        deg = 1.0 + c0_ref[:, 0:1] + c1_ref[:, 0:1]
        dis = lax.rsqrt(deg)
        y_ref[...] = jnp.dot(x_ref[...], w_ref[...],
                             preferred_element_type=jnp.float32) * dis
        dis_ref[...] = dis

    return pl.pallas_call(
        body,
        grid=(N // _R,),
        in_specs=[
            pl.BlockSpec((_R, H), lambda i: (i, 0)),
            pl.BlockSpec((H, H), lambda i: (0, 0)),
            pl.BlockSpec((_R, CW), lambda i: (i, 0)),
            pl.BlockSpec((_R, CW), lambda i: (i, 0)),
        ],
        out_specs=[
            pl.BlockSpec((_R, H), lambda i: (i, 0)),
            pl.BlockSpec((_R, 1), lambda i: (i, 0)),
        ],
        out_shape=[
            jax.ShapeDtypeStruct((N, H), jnp.float32),
            jax.ShapeDtypeStruct((N, 1), jnp.float32),
        ],
    )(x, W1, cnt0, cnt1)


def _tc_layer(s0, s1, y, dis, b, W):
    def body(s0_ref, s1_ref, y_ref, dis_ref, b_ref, w_ref, o_ref):
        dis = dis_ref[...]
        h = dis * (s0_ref[...] + s1_ref[...] + y_ref[...]) + b_ref[...]
        h = jnp.maximum(h, 0.0)
        o_ref[...] = jnp.dot(h, w_ref[...],
                             preferred_element_type=jnp.float32) * dis

    return pl.pallas_call(
        body,
        grid=(N // _R,),
        in_specs=[
            pl.BlockSpec((_R, H), lambda i: (i, 0)),
            pl.BlockSpec((_R, H), lambda i: (i, 0)),
            pl.BlockSpec((_R, H), lambda i: (i, 0)),
            pl.BlockSpec((_R, 1), lambda i: (i, 0)),
            pl.BlockSpec((1, H), lambda i: (0, 0)),
            pl.BlockSpec((H, H), lambda i: (0, 0)),
        ],
        out_specs=pl.BlockSpec((_R, H), lambda i: (i, 0)),
        out_shape=jax.ShapeDtypeStruct((N, H), jnp.float32),
    )(s0, s1, y, dis, b.reshape(1, H), W)


def _tc_final(s0, s1, y, dis, b5, batch, Wh1, bh1, Wh2, bh2):
    NHID = Wh1.shape[1]
    NOUT = Wh2.shape[1]

    def body(s0_ref, s1_ref, y_ref, dis_ref, b_ref, batch_ref,
             wh1_ref, bh1_ref, wh2_ref, bh2_ref, o_ref, psum, cnt):
        i = pl.program_id(0)

        @pl.when(i == 0)
        def _():
            psum[...] = jnp.zeros_like(psum)
            cnt[...] = jnp.zeros_like(cnt)

        h = dis_ref[...] * (s0_ref[...] + s1_ref[...] + y_ref[...]) + b_ref[...]
        gids = lax.broadcasted_iota(jnp.int32, (G, 1), 0)
        seg = jnp.where(batch_ref[...].reshape(1, _R) == gids, 1.0, 0.0)
        psum[...] += jnp.dot(seg, h, preferred_element_type=jnp.float32)
        cnt[...] += jnp.sum(seg, axis=1, keepdims=True)

        @pl.when(i == pl.num_programs(0) - 1)
        def _():
            pooled = psum[...] / jnp.maximum(cnt[...], 1.0)
            z = jnp.dot(pooled, wh1_ref[...],
                        preferred_element_type=jnp.float32) + bh1_ref[...]
            z = jnp.maximum(z, 0.0)
            o_ref[...] = jnp.dot(z, wh2_ref[...],
                                 preferred_element_type=jnp.float32) + bh2_ref[...]

    return pl.pallas_call(
        body,
        grid=(N // _R,),
        in_specs=[
            pl.BlockSpec((_R, H), lambda i: (i, 0)),
            pl.BlockSpec((_R, H), lambda i: (i, 0)),
            pl.BlockSpec((_R, H), lambda i: (i, 0)),
            pl.BlockSpec((_R, 1), lambda i: (i, 0)),
            pl.BlockSpec((1, H), lambda i: (0, 0)),
            pl.BlockSpec((_R, 1), lambda i: (i, 0)),
            pl.BlockSpec((H, NHID), lambda i: (0, 0)),
            pl.BlockSpec((1, NHID), lambda i: (0, 0)),
            pl.BlockSpec((NHID, NOUT), lambda i: (0, 0)),
            pl.BlockSpec((1, NOUT), lambda i: (0, 0)),
        ],
        out_specs=pl.BlockSpec((G, NOUT), lambda i: (0, 0)),
        out_shape=jax.ShapeDtypeStruct((G, NOUT), jnp.float32),
        scratch_shapes=[
            pltpu.VMEM((G, NOUT), jnp.float32),
            pltpu.VMEM((G, 1), jnp.float32),
        ],
    )(s0, s1, y, dis, b5.reshape(1, H), batch,
      Wh1, bh1.reshape(1, NHID), Wh2, bh2.reshape(1, NOUT))


def kernel(x, edge_index, batch, W1, b1, W2, b2, W3, b3, W4, b4, W5, b5,
           Wh1, bh1, Wh2, bh2):
    ei = edge_index.reshape(2, NW, CH, C)
    cnt = _sc_degree(ei)
    y, dis = _tc_prologue(x, W1, cnt[0], cnt[1])
    s = _sc_scatter(y, ei)
    y = _tc_layer(s[0], s[1], y, dis, b1, W2)
    s = _sc_scatter(y, ei)
    y = _tc_layer(s[0], s[1], y, dis, b2, W3)
    s = _sc_scatter(y, ei)
    y = _tc_layer(s[0], s[1], y, dis, b3, W4)
    s = _sc_scatter(y, ei)
    y = _tc_layer(s[0], s[1], y, dis, b4, W5)
    s = _sc_scatter(y, ei)
    return _tc_final(s[0], s[1], y, dis, b5, batch.reshape(N, 1),
                     Wh1, bh1, Wh2, bh2)


# speedup vs baseline: 18.7934x; 1.4465x over previous
import functools

import jax
import jax.numpy as jnp
from jax import lax
from jax.experimental import pallas as pl
from jax.experimental.pallas import tpu as pltpu
from jax.experimental.pallas import tpu_sc as plsc

N = 10000
E = 320000
H = 128
G = 64

NC = 2
NS = 16
NW = NC * NS
EPW = E // NW
C = 80
CH = EPW // C
PC = 25
PH = CH // PC
RPT = 624
TAIL = N - NS * RPT
ZR = 48
CW = 128

_mesh = plsc.VectorSubcoreMesh(core_axis_name="c", subcore_axis_name="s")


def _sc_degree(ei):
    @functools.partial(
        pl.kernel,
        out_type=jax.ShapeDtypeStruct((NC, N, CW), jnp.float32),
        mesh=_mesh,
        scratch_types=[
            pltpu.VMEM((PH, PC, C), jnp.int32),
            pltpu.VMEM((C, CW), jnp.float32),
            pltpu.VMEM((ZR, CW), jnp.float32),
            pltpu.VMEM_SHARED((N, CW), jnp.float32),
        ],
    )
    def k(ei_hbm, out_hbm, didx, ones_v, zbuf, acc):
        c = lax.axis_index("c")
        s = lax.axis_index("s")
        wid = c * NS + s
        one16 = jnp.ones((16,), jnp.float32)
        zero16 = jnp.zeros((16,), jnp.float32)

        def fill(i, _):
            for j in range(CW // 16):
                ones_v[i, pl.ds(j * 16, 16)] = one16
            return 0

        lax.fori_loop(0, C, fill, 0)

        def zrow(i, _):
            for j in range(CW // 16):
                zbuf[i, pl.ds(j * 16, 16)] = zero16
            return 0

        lax.fori_loop(0, ZR, zrow, 0)
        for r in range(RPT // ZR):
            pltpu.sync_copy(zbuf, acc.at[pl.ds(s * RPT + r * ZR, ZR)])

        @pl.when(s == NS - 1)
        def _():
            pltpu.sync_copy(zbuf.at[pl.ds(0, TAIL)],
                            acc.at[pl.ds(NS * RPT, TAIL)])

        pltpu.sync_copy(ei_hbm.at[1, wid], didx)
        plsc.subcore_barrier()

        def body(p, _):
            def inner(j, _):
                pltpu.sync_copy(ones_v, acc.at[didx.at[p, j]], add=True)
                return 0

            lax.fori_loop(0, PC, inner, 0)
            return 0

        lax.fori_loop(0, PH, body, 0)
        plsc.subcore_barrier()
        pltpu.sync_copy(acc.at[pl.ds(s * RPT, RPT)],
                        out_hbm.at[c, pl.ds(s * RPT, RPT)])

        @pl.when(s == NS - 1)
        def _():
            pltpu.sync_copy(acc.at[pl.ds(NS * RPT, TAIL)],
                            out_hbm.at[c, pl.ds(NS * RPT, TAIL)])

    return k(ei)


def _sc_scatter(y, ei):
    @functools.partial(
        pl.kernel,
        out_type=jax.ShapeDtypeStruct((NC, N, H), jnp.float32),
        mesh=_mesh,
        scratch_types=[
            pltpu.VMEM((PC, C), jnp.int32),
            pltpu.VMEM((PC, C), jnp.int32),
            pltpu.VMEM((C, H), jnp.float32),
            pltpu.VMEM((C, H), jnp.float32),
            pltpu.VMEM((ZR, H), jnp.float32),
            pltpu.VMEM_SHARED((N, H), jnp.float32),
            pltpu.SemaphoreType.DMA,
            pltpu.SemaphoreType.DMA,
        ],
    )
    def k(y_hbm, ei_hbm, out_hbm, sidx, didx, rows0, rows1, zbuf, acc,
          sem0, sem1):
        c = lax.axis_index("c")
        s = lax.axis_index("s")
        wid = c * NS + s
        zero16 = jnp.zeros((16,), jnp.float32)

        def zrow(i, _):
            for j in range(H // 16):
                zbuf[i, pl.ds(j * 16, 16)] = zero16
            return 0

        lax.fori_loop(0, ZR, zrow, 0)
        for r in range(RPT // ZR):
            pltpu.sync_copy(zbuf, acc.at[pl.ds(s * RPT + r * ZR, ZR)])

        @pl.when(s == NS - 1)
        def _():
            pltpu.sync_copy(zbuf.at[pl.ds(0, TAIL)],
                            acc.at[pl.ds(NS * RPT, TAIL)])

        plsc.subcore_barrier()

        for p in range(PH):
            pltpu.sync_copy(ei_hbm.at[0, wid, p], sidx)
            pltpu.sync_copy(ei_hbm.at[1, wid, p], didx)
            pltpu.async_copy(y_hbm.at[sidx.at[0]], rows0, sem0)

            def body(t, _):
                j = 2 * t
                pltpu.async_copy(y_hbm.at[sidx.at[j + 1]], rows1, sem1)
                pltpu.make_async_copy(y_hbm.at[sidx.at[j]], rows0, sem0).wait()
                pltpu.sync_copy(rows0, acc.at[didx.at[j]], add=True)
                pltpu.async_copy(y_hbm.at[sidx.at[j + 2]], rows0, sem0)
                pltpu.make_async_copy(y_hbm.at[sidx.at[j + 1]], rows1,
                                      sem1).wait()
                pltpu.sync_copy(rows1, acc.at[didx.at[j + 1]], add=True)
                return 0

            lax.fori_loop(0, (PC - 1) // 2, body, 0)
            pltpu.make_async_copy(y_hbm.at[sidx.at[PC - 1]], rows0,
                                  sem0).wait()
            pltpu.sync_copy(rows0, acc.at[didx.at[PC - 1]], add=True)
        plsc.subcore_barrier()
        pltpu.sync_copy(acc.at[pl.ds(s * RPT, RPT)],
                        out_hbm.at[c, pl.ds(s * RPT, RPT)])

        @pl.when(s == NS - 1)
        def _():
            pltpu.sync_copy(acc.at[pl.ds(NS * RPT, TAIL)],
                            out_hbm.at[c, pl.ds(NS * RPT, TAIL)])

    return k(y, ei)


_R = 2000


def _tc_prologue(x, W1, cnt0, cnt1):
    def body(x_ref, w_ref, c0_ref, c1_ref, y_ref, dis_ref):
        deg = 1.0 + c0_ref[:, 0:1] + c1_ref[:, 0:1]
        dis = lax.rsqrt(deg)
        y_ref[...] = jnp.dot(x_ref[...], w_ref[...],
                             preferred_element_type=jnp.float32) * dis
        dis_ref[...] = dis

    return pl.pallas_call(
        body,
        grid=(N // _R,),
        in_specs=[
            pl.BlockSpec((_R, H), lambda i: (i, 0)),
            pl.BlockSpec((H, H), lambda i: (0, 0)),
            pl.BlockSpec((_R, CW), lambda i: (i, 0)),
            pl.BlockSpec((_R, CW), lambda i: (i, 0)),
        ],
        out_specs=[
            pl.BlockSpec((_R, H), lambda i: (i, 0)),
            pl.BlockSpec((_R, 1), lambda i: (i, 0)),
        ],
        out_shape=[
            jax.ShapeDtypeStruct((N, H), jnp.float32),
            jax.ShapeDtypeStruct((N, 1), jnp.float32),
        ],
    )(x, W1, cnt0, cnt1)


def _tc_layer(s0, s1, y, dis, b, W):
    def body(s0_ref, s1_ref, y_ref, dis_ref, b_ref, w_ref, o_ref):
        dis = dis_ref[...]
        h = dis * (s0_ref[...] + s1_ref[...] + y_ref[...]) + b_ref[...]
        h = jnp.maximum(h, 0.0)
        o_ref[...] = jnp.dot(h, w_ref[...],
                             preferred_element_type=jnp.float32) * dis

    return pl.pallas_call(
        body,
        grid=(N // _R,),
        in_specs=[
            pl.BlockSpec((_R, H), lambda i: (i, 0)),
            pl.BlockSpec((_R, H), lambda i: (i, 0)),
            pl.BlockSpec((_R, H), lambda i: (i, 0)),
            pl.BlockSpec((_R, 1), lambda i: (i, 0)),
            pl.BlockSpec((1, H), lambda i: (0, 0)),
            pl.BlockSpec((H, H), lambda i: (0, 0)),
        ],
        out_specs=pl.BlockSpec((_R, H), lambda i: (i, 0)),
        out_shape=jax.ShapeDtypeStruct((N, H), jnp.float32),
    )(s0, s1, y, dis, b.reshape(1, H), W)


def _tc_final(s0, s1, y, dis, b5, batch, Wh1, bh1, Wh2, bh2):
    NHID = Wh1.shape[1]
    NOUT = Wh2.shape[1]

    def body(s0_ref, s1_ref, y_ref, dis_ref, b_ref, batch_ref,
             wh1_ref, bh1_ref, wh2_ref, bh2_ref, o_ref, psum, cnt):
        i = pl.program_id(0)

        @pl.when(i == 0)
        def _():
            psum[...] = jnp.zeros_like(psum)
            cnt[...] = jnp.zeros_like(cnt)

        h = dis_ref[...] * (s0_ref[...] + s1_ref[...] + y_ref[...]) + b_ref[...]
        gids = lax.broadcasted_iota(jnp.int32, (G, 1), 0)
        seg = jnp.where(batch_ref[...].reshape(1, _R) == gids, 1.0, 0.0)
        psum[...] += jnp.dot(seg, h, preferred_element_type=jnp.float32)
        cnt[...] += jnp.sum(seg, axis=1, keepdims=True)

        @pl.when(i == pl.num_programs(0) - 1)
        def _():
            pooled = psum[...] / jnp.maximum(cnt[...], 1.0)
            z = jnp.dot(pooled, wh1_ref[...],
                        preferred_element_type=jnp.float32) + bh1_ref[...]
            z = jnp.maximum(z, 0.0)
            o_ref[...] = jnp.dot(z, wh2_ref[...],
                                 preferred_element_type=jnp.float32) + bh2_ref[...]

    return pl.pallas_call(
        body,
        grid=(N // _R,),
        in_specs=[
            pl.BlockSpec((_R, H), lambda i: (i, 0)),
            pl.BlockSpec((_R, H), lambda i: (i, 0)),
            pl.BlockSpec((_R, H), lambda i: (i, 0)),
            pl.BlockSpec((_R, 1), lambda i: (i, 0)),
            pl.BlockSpec((1, H), lambda i: (0, 0)),
            pl.BlockSpec((_R, 1), lambda i: (i, 0)),
            pl.BlockSpec((H, NHID), lambda i: (0, 0)),
            pl.BlockSpec((1, NHID), lambda i: (0, 0)),
            pl.BlockSpec((NHID, NOUT), lambda i: (0, 0)),
            pl.BlockSpec((1, NOUT), lambda i: (0, 0)),
        ],
        out_specs=pl.BlockSpec((G, NOUT), lambda i: (0, 0)),
        out_shape=jax.ShapeDtypeStruct((G, NOUT), jnp.float32),
        scratch_shapes=[
            pltpu.VMEM((G, NOUT), jnp.float32),
            pltpu.VMEM((G, 1), jnp.float32),
        ],
    )(s0, s1, y, dis, b5.reshape(1, H), batch,
      Wh1, bh1.reshape(1, NHID), Wh2, bh2.reshape(1, NOUT))


def kernel(x, edge_index, batch, W1, b1, W2, b2, W3, b3, W4, b4, W5, b5,
           Wh1, bh1, Wh2, bh2):
    ei = edge_index.reshape(2, NW, PH, PC, C)
    cnt = _sc_degree(ei)
    y, dis = _tc_prologue(x, W1, cnt[0], cnt[1])
    s = _sc_scatter(y, ei)
    y = _tc_layer(s[0], s[1], y, dis, b1, W2)
    s = _sc_scatter(y, ei)
    y = _tc_layer(s[0], s[1], y, dis, b2, W3)
    s = _sc_scatter(y, ei)
    y = _tc_layer(s[0], s[1], y, dis, b3, W4)
    s = _sc_scatter(y, ei)
    y = _tc_layer(s[0], s[1], y, dis, b4, W5)
    s = _sc_scatter(y, ei)
    return _tc_final(s[0], s[1], y, dis, b5, batch.reshape(N, 1),
                     Wh1, bh1, Wh2, bh2)


# speedup vs baseline: 20.5948x; 1.0959x over previous
import functools

import jax
import jax.numpy as jnp
from jax import lax
from jax.experimental import pallas as pl
from jax.experimental.pallas import tpu as pltpu
from jax.experimental.pallas import tpu_sc as plsc

N = 10000
E = 320000
H = 128
G = 64

NC = 2
NS = 16
NW = NC * NS
EPW = E // NW
C = 80
CH = EPW // C
PC = 25
PH = CH // PC
RPT = 624
TAIL = N - NS * RPT
ZR = 16
CW = 128

_mesh = plsc.VectorSubcoreMesh(core_axis_name="c", subcore_axis_name="s")


def _sc_degree(ei):
    @functools.partial(
        pl.kernel,
        out_type=jax.ShapeDtypeStruct((NC, N, CW), jnp.float32),
        mesh=_mesh,
        scratch_types=[
            pltpu.VMEM((PH, PC, C), jnp.int32),
            pltpu.VMEM((C, CW), jnp.float32),
            pltpu.VMEM((ZR, CW), jnp.float32),
            pltpu.VMEM_SHARED((N, CW), jnp.float32),
            pltpu.SemaphoreType.DMA,
        ],
    )
    def k(ei_hbm, out_hbm, didx, ones_v, zbuf, acc, dsem):
        c = lax.axis_index("c")
        s = lax.axis_index("s")
        wid = c * NS + s
        one16 = jnp.ones((16,), jnp.float32)
        zero16 = jnp.zeros((16,), jnp.float32)

        def fill(i, _):
            for j in range(CW // 16):
                ones_v[i, pl.ds(j * 16, 16)] = one16
            return 0

        lax.fori_loop(0, C, fill, 0)

        def zrow(i, _):
            for j in range(CW // 16):
                zbuf[i, pl.ds(j * 16, 16)] = zero16
            return 0

        lax.fori_loop(0, ZR, zrow, 0)
        for r in range(RPT // ZR):
            pltpu.sync_copy(zbuf, acc.at[pl.ds(s * RPT + r * ZR, ZR)])

        @pl.when(s == NS - 1)
        def _():
            pltpu.sync_copy(zbuf.at[pl.ds(0, TAIL)],
                            acc.at[pl.ds(NS * RPT, TAIL)])

        pltpu.sync_copy(ei_hbm.at[1, wid], didx)
        plsc.subcore_barrier()

        def body(p, _):
            def fire(j, _):
                pltpu.async_copy(ones_v, acc.at[didx.at[p, j]], dsem,
                                 add=True)
                return 0

            lax.fori_loop(0, PC, fire, 0)

            def drain(j, _):
                pltpu.make_async_copy(ones_v, acc.at[didx.at[p, j]],
                                      dsem).wait()
                return 0

            lax.fori_loop(0, PC, drain, 0)
            return 0

        lax.fori_loop(0, PH, body, 0)
        plsc.subcore_barrier()
        pltpu.sync_copy(acc.at[pl.ds(s * RPT, RPT)],
                        out_hbm.at[c, pl.ds(s * RPT, RPT)])

        @pl.when(s == NS - 1)
        def _():
            pltpu.sync_copy(acc.at[pl.ds(NS * RPT, TAIL)],
                            out_hbm.at[c, pl.ds(NS * RPT, TAIL)])

    return k(ei)


def _sc_scatter(y, ei):
    @functools.partial(
        pl.kernel,
        out_type=jax.ShapeDtypeStruct((NC, N, H), jnp.float32),
        mesh=_mesh,
        scratch_types=[
            pltpu.VMEM((PC, C), jnp.int32),
            pltpu.VMEM((PC, C), jnp.int32),
            pltpu.VMEM((C, H), jnp.float32),
            pltpu.VMEM((C, H), jnp.float32),
            pltpu.VMEM((C, H), jnp.float32),
            pltpu.VMEM((ZR, H), jnp.float32),
            pltpu.VMEM_SHARED((N, H), jnp.float32),
            pltpu.SemaphoreType.DMA,
            pltpu.SemaphoreType.DMA,
            pltpu.SemaphoreType.DMA,
            pltpu.SemaphoreType.DMA,
            pltpu.SemaphoreType.DMA,
            pltpu.SemaphoreType.DMA,
        ],
    )
    def k(y_hbm, ei_hbm, out_hbm, sidx, didx, rows0, rows1, rows2, zbuf, acc,
          semg0, semg1, semg2, sems0, sems1, sems2):
        c = lax.axis_index("c")
        s = lax.axis_index("s")
        wid = c * NS + s
        zero16 = jnp.zeros((16,), jnp.float32)

        def zrow(i, _):
            for j in range(H // 16):
                zbuf[i, pl.ds(j * 16, 16)] = zero16
            return 0

        lax.fori_loop(0, ZR, zrow, 0)
        for r in range(RPT // ZR):
            pltpu.sync_copy(zbuf, acc.at[pl.ds(s * RPT + r * ZR, ZR)])

        @pl.when(s == NS - 1)
        def _():
            pltpu.sync_copy(zbuf.at[pl.ds(0, TAIL)],
                            acc.at[pl.ds(NS * RPT, TAIL)])

        plsc.subcore_barrier()

        rows = (rows0, rows1, rows2)
        semg = (semg0, semg1, semg2)
        sems = (sems0, sems1, sems2)

        def gath(j, b):
            pltpu.async_copy(y_hbm.at[sidx.at[j]], rows[b], semg[b])

        def gath_wait(j, b):
            pltpu.make_async_copy(y_hbm.at[sidx.at[j]], rows[b],
                                  semg[b]).wait()

        def scat(j, b):
            pltpu.async_copy(rows[b], acc.at[didx.at[j]], sems[b], add=True)

        def scat_wait(j, b):
            pltpu.make_async_copy(rows[b], acc.at[didx.at[j]], sems[b]).wait()

        for p in range(PH):
            pltpu.sync_copy(ei_hbm.at[0, wid, p], sidx)
            pltpu.sync_copy(ei_hbm.at[1, wid, p], didx)
            gath(0, 0)
            gath(1, 1)

            def body(t, _):
                for b in range(3):
                    j = 3 * t + b
                    gath_wait(j, b)
                    scat(j, b)

                    @pl.when(j >= 1)
                    def _():
                        scat_wait(j - 1, (b + 2) % 3)

                    @pl.when(j + 2 <= PC - 1)
                    def _():
                        gath(j + 2, (b + 2) % 3)
                return 0

            lax.fori_loop(0, (PC - 1) // 3, body, 0)
            gath_wait(PC - 1, (PC - 1) % 3)
            scat(PC - 1, (PC - 1) % 3)
            for j in (PC - 2, PC - 1):
                scat_wait(j, j % 3)
        plsc.subcore_barrier()
        pltpu.sync_copy(acc.at[pl.ds(s * RPT, RPT)],
                        out_hbm.at[c, pl.ds(s * RPT, RPT)])

        @pl.when(s == NS - 1)
        def _():
            pltpu.sync_copy(acc.at[pl.ds(NS * RPT, TAIL)],
                            out_hbm.at[c, pl.ds(NS * RPT, TAIL)])

    return k(y, ei)


_R = 2000


def _tc_prologue(x, W1, cnt0, cnt1):
    def body(x_ref, w_ref, c0_ref, c1_ref, y_ref, dis_ref):
        deg = 1.0 + c0_ref[:, 0:1] + c1_ref[:, 0:1]
        dis = lax.rsqrt(deg)
        y_ref[...] = jnp.dot(x_ref[...], w_ref[...],
                             preferred_element_type=jnp.float32) * dis
        dis_ref[...] = dis

    return pl.pallas_call(
        body,
        grid=(N // _R,),
        in_specs=[
            pl.BlockSpec((_R, H), lambda i: (i, 0)),
            pl.BlockSpec((H, H), lambda i: (0, 0)),
            pl.BlockSpec((_R, CW), lambda i: (i, 0)),
            pl.BlockSpec((_R, CW), lambda i: (i, 0)),
        ],
        out_specs=[
            pl.BlockSpec((_R, H), lambda i: (i, 0)),
            pl.BlockSpec((_R, 1), lambda i: (i, 0)),
        ],
        out_shape=[
            jax.ShapeDtypeStruct((N, H), jnp.float32),
            jax.ShapeDtypeStruct((N, 1), jnp.float32),
        ],
    )(x, W1, cnt0, cnt1)


def _tc_layer(s0, s1, y, dis, b, W):
    def body(s0_ref, s1_ref, y_ref, dis_ref, b_ref, w_ref, o_ref):
        dis = dis_ref[...]
        h = dis * (s0_ref[...] + s1_ref[...] + y_ref[...]) + b_ref[...]
        h = jnp.maximum(h, 0.0)
        o_ref[...] = jnp.dot(h, w_ref[...],
                             preferred_element_type=jnp.float32) * dis

    return pl.pallas_call(
        body,
        grid=(N // _R,),
        in_specs=[
            pl.BlockSpec((_R, H), lambda i: (i, 0)),
            pl.BlockSpec((_R, H), lambda i: (i, 0)),
            pl.BlockSpec((_R, H), lambda i: (i, 0)),
            pl.BlockSpec((_R, 1), lambda i: (i, 0)),
            pl.BlockSpec((1, H), lambda i: (0, 0)),
            pl.BlockSpec((H, H), lambda i: (0, 0)),
        ],
        out_specs=pl.BlockSpec((_R, H), lambda i: (i, 0)),
        out_shape=jax.ShapeDtypeStruct((N, H), jnp.float32),
    )(s0, s1, y, dis, b.reshape(1, H), W)


def _tc_final(s0, s1, y, dis, b5, batch, Wh1, bh1, Wh2, bh2):
    NHID = Wh1.shape[1]
    NOUT = Wh2.shape[1]

    def body(s0_ref, s1_ref, y_ref, dis_ref, b_ref, batch_ref,
             wh1_ref, bh1_ref, wh2_ref, bh2_ref, o_ref, psum, cnt):
        i = pl.program_id(0)

        @pl.when(i == 0)
        def _():
            psum[...] = jnp.zeros_like(psum)
            cnt[...] = jnp.zeros_like(cnt)

        h = dis_ref[...] * (s0_ref[...] + s1_ref[...] + y_ref[...]) + b_ref[...]
        gids = lax.broadcasted_iota(jnp.int32, (G, 1), 0)
        seg = jnp.where(batch_ref[...].reshape(1, _R) == gids, 1.0, 0.0)
        psum[...] += jnp.dot(seg, h, preferred_element_type=jnp.float32)
        cnt[...] += jnp.sum(seg, axis=1, keepdims=True)

        @pl.when(i == pl.num_programs(0) - 1)
        def _():
            pooled = psum[...] / jnp.maximum(cnt[...], 1.0)
            z = jnp.dot(pooled, wh1_ref[...],
                        preferred_element_type=jnp.float32) + bh1_ref[...]
            z = jnp.maximum(z, 0.0)
            o_ref[...] = jnp.dot(z, wh2_ref[...],
                                 preferred_element_type=jnp.float32) + bh2_ref[...]

    return pl.pallas_call(
        body,
        grid=(N // _R,),
        in_specs=[
            pl.BlockSpec((_R, H), lambda i: (i, 0)),
            pl.BlockSpec((_R, H), lambda i: (i, 0)),
            pl.BlockSpec((_R, H), lambda i: (i, 0)),
            pl.BlockSpec((_R, 1), lambda i: (i, 0)),
            pl.BlockSpec((1, H), lambda i: (0, 0)),
            pl.BlockSpec((_R, 1), lambda i: (i, 0)),
            pl.BlockSpec((H, NHID), lambda i: (0, 0)),
            pl.BlockSpec((1, NHID), lambda i: (0, 0)),
            pl.BlockSpec((NHID, NOUT), lambda i: (0, 0)),
            pl.BlockSpec((1, NOUT), lambda i: (0, 0)),
        ],
        out_specs=pl.BlockSpec((G, NOUT), lambda i: (0, 0)),
        out_shape=jax.ShapeDtypeStruct((G, NOUT), jnp.float32),
        scratch_shapes=[
            pltpu.VMEM((G, NOUT), jnp.float32),
            pltpu.VMEM((G, 1), jnp.float32),
        ],
    )(s0, s1, y, dis, b5.reshape(1, H), batch,
      Wh1, bh1.reshape(1, NHID), Wh2, bh2.reshape(1, NOUT))


def kernel(x, edge_index, batch, W1, b1, W2, b2, W3, b3, W4, b4, W5, b5,
           Wh1, bh1, Wh2, bh2):
    ei = edge_index.reshape(2, NW, PH, PC, C)
    cnt = _sc_degree(ei)
    y, dis = _tc_prologue(x, W1, cnt[0], cnt[1])
    s = _sc_scatter(y, ei)
    y = _tc_layer(s[0], s[1], y, dis, b1, W2)
    s = _sc_scatter(y, ei)
    y = _tc_layer(s[0], s[1], y, dis, b2, W3)
    s = _sc_scatter(y, ei)
    y = _tc_layer(s[0], s[1], y, dis, b3, W4)
    s = _sc_scatter(y, ei)
    y = _tc_layer(s[0], s[1], y, dis, b4, W5)
    s = _sc_scatter(y, ei)
    return _tc_final(s[0], s[1], y, dis, b5, batch.reshape(N, 1),
                     Wh1, bh1, Wh2, bh2)
